# baseline (speedup 1.0000x reference)
import functools

import jax
import jax.numpy as jnp
from jax import lax
from jax.experimental import pallas as pl
from jax.experimental.pallas import tpu as pltpu
from jax.experimental.pallas import tpu_sc as plsc

N = 10000
E = 320000
D = 128
H = 32
C = 2
CP = 8

NC = 2
NS = 16
NW = NC * NS
CH = 128
NCH = 80
KB = 8
F0 = 160
F1 = 2 * NCH - F0
SMAX = max(F0, F1) if min(F0, F1) > 0 else max(F0, F1)
EP = NW * NCH * CH
NPAD = 10240
RPT = NPAD // NS

_mesh = lambda: plsc.VectorSubcoreMesh(core_axis_name="c", subcore_axis_name="s")


def _make_propagate(width):
    @functools.partial(
        pl.kernel,
        mesh=_mesh(),
        out_type=jax.ShapeDtypeStruct((NC, NPAD, width), jnp.float32),
        compiler_params=pltpu.CompilerParams(use_tc_tiling_on_sc=False),
        scratch_types=[
            pltpu.VMEM((SMAX, CH), jnp.int32),
            pltpu.VMEM((SMAX, CH), jnp.int32),
        ]
        + [pltpu.VMEM((CH, width), jnp.float32) for _ in range(KB)]
        + [pltpu.VMEM_SHARED((NPAD, width), jnp.float32)]
        + [pltpu.SemaphoreType.DMA for _ in range(KB)],
    )
    def propagate(feat, src2d, dst2d, zeros, out, sidx, didx, *rest):
        rows = rest[:KB]
        acc = rest[KB]
        sems = rest[KB + 1:]
        c = lax.axis_index("c")
        s = lax.axis_index("s")
        pltpu.sync_copy(zeros, acc.at[pl.ds(s * RPT, RPT)])
        plsc.subcore_barrier()

        def block(j, carry):
            hs = [pltpu.async_copy(feat.at[sidx.at[j + b]], rows[b], sems[b])
                  for b in range(KB)]
            for b in range(KB):
                hs[b].wait()
                pltpu.sync_copy(rows[b], acc.at[didx.at[j + b]], add=True)
            return carry

        if F0 > 0:
            @pl.when(c == 0)
            def _():
                pltpu.sync_copy(src2d.at[pl.ds(s * F0, F0)],
                                sidx.at[pl.ds(0, F0)])
                pltpu.sync_copy(dst2d.at[pl.ds(s * F0, F0)],
                                didx.at[pl.ds(0, F0)])
                lax.fori_loop(0, F0 // KB, lambda k, cy: block(k * KB, cy), 0)

        if F1 > 0:
            @pl.when(c != 0)
            def _():
                pltpu.sync_copy(src2d.at[pl.ds(NS * F0 + s * F1, F1)],
                                sidx.at[pl.ds(0, F1)])
                pltpu.sync_copy(dst2d.at[pl.ds(NS * F0 + s * F1, F1)],
                                didx.at[pl.ds(0, F1)])
                lax.fori_loop(0, F1 // KB, lambda k, cy: block(k * KB, cy), 0)

        plsc.subcore_barrier()
        pltpu.sync_copy(acc.at[pl.ds(s * RPT, RPT)],
                        out.at[c, pl.ds(s * RPT, RPT)])

    return propagate


_propagate_h = _make_propagate(H)
_propagate_c = _make_propagate(CP)


@functools.partial(
    pl.kernel,
    mesh=_mesh(),
    out_type=jax.ShapeDtypeStruct((NC, NPAD), jnp.float32),
    compiler_params=pltpu.CompilerParams(use_tc_tiling_on_sc=False),
    scratch_types=[
        pltpu.VMEM((NCH, CH), jnp.int32),
        pltpu.VMEM((CH,), jnp.float32),
        pltpu.VMEM_SHARED((NPAD,), jnp.float32),
    ],
)
def _sc_degree(dst2d, zeros1, ones1, out, didx, ones_v, acc):
    c = lax.axis_index("c")
    s = lax.axis_index("s")
    wid = s * NC + c
    pltpu.sync_copy(zeros1, acc.at[pl.ds(s * RPT, RPT)])
    pltpu.sync_copy(dst2d.at[pl.ds(wid * NCH, NCH)], didx)
    pltpu.sync_copy(ones1, ones_v)
    plsc.subcore_barrier()

    def chunk(j, carry):
        pltpu.sync_copy(ones_v, acc.at[didx.at[j]], add=True)
        return carry

    lax.fori_loop(0, NCH, chunk, 0)
    plsc.subcore_barrier()
    pltpu.sync_copy(acc.at[pl.ds(s * RPT, RPT)], out.at[c, pl.ds(s * RPT, RPT)])


def _tc_mm1(x, W1):
    def body(x_ref, w1_ref, h1_ref):
        h1_ref[...] = jnp.dot(x_ref[...], w1_ref[...],
                              preferred_element_type=jnp.float32)

    return pl.pallas_call(
        body,
        out_shape=jax.ShapeDtypeStruct((N, H), jnp.float32),
    )(x, W1)


def _tc_scale(degp3, h1):
    def body(degp_ref, h1_ref, h1s_ref, dinv_ref):
        deg = degp_ref[0] + degp_ref[1] + 1.0
        dinv = 1.0 / jnp.sqrt(deg)
        dinv_ref[...] = dinv
        h1s_ref[...] = h1_ref[...] * dinv[:N]

    return pl.pallas_call(
        body,
        out_shape=(jax.ShapeDtypeStruct((N, H), jnp.float32),
                   jax.ShapeDtypeStruct((NPAD, 1), jnp.float32)),
    )(degp3, h1)


def _tc_mm2(p, h1s, dinv, b1, W2):
    def body(p_ref, h1s_ref, dinv_ref, b1_ref, w2_ref, g_ref):
        dinv = dinv_ref[...][:N]
        t = p_ref[0, :N, :] + p_ref[1, :N, :] + h1s_ref[...]
        h = jnp.maximum(t * dinv + b1_ref[...][None, :], 0.0)
        g_ref[...] = jnp.dot(h, w2_ref[...],
                             preferred_element_type=jnp.float32) * dinv

    return pl.pallas_call(
        body,
        out_shape=jax.ShapeDtypeStruct((N, CP), jnp.float32),
    )(p, h1s, dinv, b1, W2)


def _tc_finish(q, g, dinv, b2):
    def body(q_ref, g_ref, dinv_ref, b2_ref, out_ref):
        dinv = dinv_ref[...][:N]
        o = (q_ref[0, :N, :] + q_ref[1, :N, :] + g_ref[...]) * dinv
        o = o[:, :C] + b2_ref[...][None, :]
        m = jnp.max(o, axis=1, keepdims=True)
        lse = jnp.log(jnp.sum(jnp.exp(o - m), axis=1, keepdims=True)) + m
        out_ref[...] = o - lse

    return pl.pallas_call(
        body,
        out_shape=jax.ShapeDtypeStruct((N, C), jnp.float32),
    )(q, g, dinv, b2)


def kernel(x, edge_index, W1, b1, W2, b2):
    pad = EP - E
    src2d = jnp.concatenate(
        [edge_index[0], jnp.zeros((pad,), jnp.int32)]).reshape(EP // CH, CH)
    dst2d = jnp.concatenate(
        [edge_index[1], jnp.full((pad,), NPAD - 1, jnp.int32)]).reshape(EP // CH, CH)
    zeros_h = jnp.zeros((RPT, H), jnp.float32)
    zeros_c = jnp.zeros((RPT, CP), jnp.float32)
    W2p = jnp.pad(W2, ((0, 0), (0, CP - C)))
    zeros_1 = jnp.zeros((RPT,), jnp.float32)
    ones_ch = jnp.ones((CH,), jnp.float32)

    h1 = _tc_mm1(x, W1)
    degp = _sc_degree(dst2d, zeros_1, ones_ch)
    h1s, dinv = _tc_scale(degp.reshape(NC, NPAD, 1), h1)
    p = _propagate_h(h1s, src2d, dst2d, zeros_h)
    g = _tc_mm2(p, h1s, dinv, b1, W2p)
    q = _propagate_c(g, src2d, dst2d, zeros_c)
    return _tc_finish(q, g, dinv, b2)


# speedup vs baseline: 1.1633x; 1.1633x over previous
import functools

import jax
import jax.numpy as jnp
from jax import lax
from jax.experimental import pallas as pl
from jax.experimental.pallas import tpu as pltpu
from jax.experimental.pallas import tpu_sc as plsc

N = 10000
E = 320000
D = 128
H = 32
C = 2
CP = 8

NC = 2
NS = 16
NW = NC * NS
CH = 128
NCH = 80
KB = 8
F0 = 80
F1 = 2 * NCH - F0
SMAX = max(F0, F1)
EP = NW * NCH * CH
NPAD = 10240
RPT = NPAD // NS

_mesh = lambda: plsc.VectorSubcoreMesh(core_axis_name="c", subcore_axis_name="s")


def _make_propagate(width):
    @functools.partial(
        pl.kernel,
        mesh=_mesh(),
        out_type=jax.ShapeDtypeStruct((NC, NPAD, width), jnp.float32),
        compiler_params=pltpu.CompilerParams(use_tc_tiling_on_sc=False),
        scratch_types=[
            pltpu.VMEM((SMAX, CH), jnp.int32),
            pltpu.VMEM((SMAX, CH), jnp.int32),
        ]
        + [pltpu.VMEM((CH, width), jnp.float32) for _ in range(KB)]
        + [pltpu.VMEM_SHARED((NPAD, width), jnp.float32)]
        + [pltpu.SemaphoreType.DMA for _ in range(KB)],
    )
    def propagate(feat, src2d, dst2d, zeros, out, sidx, didx, *rest):
        rows = rest[:KB]
        acc = rest[KB]
        sems = rest[KB + 1:]
        c = lax.axis_index("c")
        s = lax.axis_index("s")
        pltpu.sync_copy(zeros, acc.at[pl.ds(s * RPT, RPT)])
        plsc.subcore_barrier()

        def block(j, carry):
            hs = [pltpu.async_copy(feat.at[sidx.at[j + b]], rows[b], sems[b])
                  for b in range(KB)]
            for b in range(KB):
                hs[b].wait()
                pltpu.sync_copy(rows[b], acc.at[didx.at[j + b]], add=True)
            return carry

        if F0 > 0:
            @pl.when(c == 0)
            def _():
                pltpu.sync_copy(src2d.at[pl.ds(s * F0, F0)],
                                sidx.at[pl.ds(0, F0)])
                pltpu.sync_copy(dst2d.at[pl.ds(s * F0, F0)],
                                didx.at[pl.ds(0, F0)])
                lax.fori_loop(0, F0 // KB, lambda k, cy: block(k * KB, cy), 0)

        if F1 > 0:
            @pl.when(c != 0)
            def _():
                pltpu.sync_copy(src2d.at[pl.ds(NS * F0 + s * F1, F1)],
                                sidx.at[pl.ds(0, F1)])
                pltpu.sync_copy(dst2d.at[pl.ds(NS * F0 + s * F1, F1)],
                                didx.at[pl.ds(0, F1)])
                lax.fori_loop(0, F1 // KB, lambda k, cy: block(k * KB, cy), 0)

        plsc.subcore_barrier()
        pltpu.sync_copy(acc.at[pl.ds(s * RPT, RPT)],
                        out.at[c, pl.ds(s * RPT, RPT)])

    return propagate


_propagate_h = _make_propagate(H)
_propagate_c = _make_propagate(CP)


@functools.partial(
    pl.kernel,
    mesh=_mesh(),
    out_type=jax.ShapeDtypeStruct((NC, NPAD), jnp.float32),
    compiler_params=pltpu.CompilerParams(use_tc_tiling_on_sc=False),
    scratch_types=[
        pltpu.VMEM((NCH, CH), jnp.int32),
        pltpu.VMEM((CH,), jnp.float32),
        pltpu.VMEM_SHARED((NPAD,), jnp.float32),
    ],
)
def _sc_degree(dst2d, zeros1, ones1, out, didx, ones_v, acc):
    c = lax.axis_index("c")
    s = lax.axis_index("s")
    wid = s * NC + c
    pltpu.sync_copy(zeros1, acc.at[pl.ds(s * RPT, RPT)])
    pltpu.sync_copy(dst2d.at[pl.ds(wid * NCH, NCH)], didx)
    pltpu.sync_copy(ones1, ones_v)
    plsc.subcore_barrier()

    def chunk(j, carry):
        pltpu.sync_copy(ones_v, acc.at[didx.at[j]], add=True)
        return carry

    lax.fori_loop(0, NCH, chunk, 0)
    plsc.subcore_barrier()
    pltpu.sync_copy(acc.at[pl.ds(s * RPT, RPT)], out.at[c, pl.ds(s * RPT, RPT)])


def _tc_mm1(x, W1):
    def body(x_ref, w1_ref, h1_ref):
        h1_ref[...] = jnp.dot(x_ref[...], w1_ref[...],
                              preferred_element_type=jnp.float32)

    return pl.pallas_call(
        body,
        out_shape=jax.ShapeDtypeStruct((N, H), jnp.float32),
    )(x, W1)


def _tc_scale(degp3, h1):
    def body(degp_ref, h1_ref, h1s_ref, dinv_ref):
        deg = degp_ref[0] + degp_ref[1] + 1.0
        dinv = 1.0 / jnp.sqrt(deg)
        dinv_ref[...] = dinv
        h1s_ref[...] = h1_ref[...] * dinv[:N]

    return pl.pallas_call(
        body,
        out_shape=(jax.ShapeDtypeStruct((N, H), jnp.float32),
                   jax.ShapeDtypeStruct((NPAD, 1), jnp.float32)),
    )(degp3, h1)


def _tc_mm2(p, h1s, dinv, b1, W2):
    def body(p_ref, h1s_ref, dinv_ref, b1_ref, w2_ref, g_ref):
        dinv = dinv_ref[...][:N]
        t = p_ref[0, :N, :] + p_ref[1, :N, :] + h1s_ref[...]
        h = jnp.maximum(t * dinv + b1_ref[...][None, :], 0.0)
        g_ref[...] = jnp.dot(h, w2_ref[...],
                             preferred_element_type=jnp.float32) * dinv

    return pl.pallas_call(
        body,
        out_shape=jax.ShapeDtypeStruct((N, CP), jnp.float32),
    )(p, h1s, dinv, b1, W2)


def _tc_finish(q, g, dinv, b2):
    def body(q_ref, g_ref, dinv_ref, b2_ref, out_ref):
        dinv = dinv_ref[...][:N]
        o = (q_ref[0, :N, :] + q_ref[1, :N, :] + g_ref[...]) * dinv
        o = o[:, :C] + b2_ref[...][None, :]
        m = jnp.max(o, axis=1, keepdims=True)
        lse = jnp.log(jnp.sum(jnp.exp(o - m), axis=1, keepdims=True)) + m
        out_ref[...] = o - lse

    return pl.pallas_call(
        body,
        out_shape=jax.ShapeDtypeStruct((N, C), jnp.float32),
    )(q, g, dinv, b2)


def kernel(x, edge_index, W1, b1, W2, b2):
    pad = EP - E
    src2d = jnp.concatenate(
        [edge_index[0], jnp.zeros((pad,), jnp.int32)]).reshape(EP // CH, CH)
    dst2d = jnp.concatenate(
        [edge_index[1], jnp.full((pad,), NPAD - 1, jnp.int32)]).reshape(EP // CH, CH)
    zeros_h = jnp.zeros((RPT, H), jnp.float32)
    zeros_c = jnp.zeros((RPT, CP), jnp.float32)
    W2p = jnp.pad(W2, ((0, 0), (0, CP - C)))
    zeros_1 = jnp.zeros((RPT,), jnp.float32)
    ones_ch = jnp.ones((CH,), jnp.float32)

    h1 = _tc_mm1(x, W1)
    degp = _sc_degree(dst2d, zeros_1, ones_ch)
    h1s, dinv = _tc_scale(degp.reshape(NC, NPAD, 1), h1)
    p = _propagate_h(h1s, src2d, dst2d, zeros_h)
    g = _tc_mm2(p, h1s, dinv, b1, W2p)
    q = _propagate_c(g, src2d, dst2d, zeros_c)
    return _tc_finish(q, g, dinv, b2)


# speedup vs baseline: 1.1780x; 1.0126x over previous
import functools

import jax
import jax.numpy as jnp
from jax import lax
from jax.experimental import pallas as pl
from jax.experimental.pallas import tpu as pltpu
from jax.experimental.pallas import tpu_sc as plsc

N = 10000
E = 320000
D = 128
H = 32
C = 2
CP = 8

NC = 2
NS = 16
NW = NC * NS
CH = 128
NCH = 80
KB = 16
F0 = 80
F1 = 2 * NCH - F0
SMAX = max(F0, F1)
EP = NW * NCH * CH
NPAD = 10240
RPT = NPAD // NS

_mesh = lambda: plsc.VectorSubcoreMesh(core_axis_name="c", subcore_axis_name="s")


def _make_propagate(width):
    @functools.partial(
        pl.kernel,
        mesh=_mesh(),
        out_type=jax.ShapeDtypeStruct((NC, NPAD, width), jnp.float32),
        compiler_params=pltpu.CompilerParams(use_tc_tiling_on_sc=False),
        scratch_types=[
            pltpu.VMEM((SMAX, CH), jnp.int32),
            pltpu.VMEM((SMAX, CH), jnp.int32),
        ]
        + [pltpu.VMEM((CH, width), jnp.float32) for _ in range(KB)]
        + [pltpu.VMEM_SHARED((NPAD, width), jnp.float32)]
        + [pltpu.SemaphoreType.DMA for _ in range(KB)],
    )
    def propagate(feat, src2d, dst2d, zeros, out, sidx, didx, *rest):
        rows = rest[:KB]
        acc = rest[KB]
        sems = rest[KB + 1:]
        c = lax.axis_index("c")
        s = lax.axis_index("s")

        @pl.when(c == 0)
        def _():
            pltpu.sync_copy(src2d.at[pl.ds(s * F0, F0)],
                            sidx.at[pl.ds(0, F0)])
            pltpu.sync_copy(dst2d.at[pl.ds(s * F0, F0)],
                            didx.at[pl.ds(0, F0)])

        @pl.when(c != 0)
        def _():
            pltpu.sync_copy(src2d.at[pl.ds(NS * F0 + s * F1, F1)],
                            sidx.at[pl.ds(0, F1)])
            pltpu.sync_copy(dst2d.at[pl.ds(NS * F0 + s * F1, F1)],
                            didx.at[pl.ds(0, F1)])

        pltpu.sync_copy(zeros, acc.at[pl.ds(s * RPT, RPT)])
        plsc.subcore_barrier()

        def block(j, carry):
            hs = [pltpu.async_copy(feat.at[sidx.at[j + b]], rows[b], sems[b])
                  for b in range(KB)]
            for b in range(KB):
                hs[b].wait()
                pltpu.sync_copy(rows[b], acc.at[didx.at[j + b]], add=True)
            return carry

        @pl.when(c == 0)
        def _():
            lax.fori_loop(0, F0 // KB, lambda k, cy: block(k * KB, cy), 0)

        @pl.when(c != 0)
        def _():
            lax.fori_loop(0, F1 // KB, lambda k, cy: block(k * KB, cy), 0)

        plsc.subcore_barrier()
        pltpu.sync_copy(acc.at[pl.ds(s * RPT, RPT)],
                        out.at[c, pl.ds(s * RPT, RPT)])

    return propagate


_propagate_h = _make_propagate(H)
_propagate_c = _make_propagate(CP)


@functools.partial(
    pl.kernel,
    mesh=_mesh(),
    out_type=jax.ShapeDtypeStruct((NC, NPAD), jnp.float32),
    compiler_params=pltpu.CompilerParams(use_tc_tiling_on_sc=False),
    scratch_types=[
        pltpu.VMEM((NCH, CH), jnp.int32),
        pltpu.VMEM((CH,), jnp.float32),
        pltpu.VMEM_SHARED((NPAD,), jnp.float32),
    ],
)
def _sc_degree(dst2d, zeros1, ones1, out, didx, ones_v, acc):
    c = lax.axis_index("c")
    s = lax.axis_index("s")
    wid = s * NC + c
    pltpu.sync_copy(zeros1, acc.at[pl.ds(s * RPT, RPT)])
    pltpu.sync_copy(dst2d.at[pl.ds(wid * NCH, NCH)], didx)
    pltpu.sync_copy(ones1, ones_v)
    plsc.subcore_barrier()

    def chunk(j, carry):
        pltpu.sync_copy(ones_v, acc.at[didx.at[j]], add=True)
        return carry

    lax.fori_loop(0, NCH, chunk, 0)
    plsc.subcore_barrier()
    pltpu.sync_copy(acc.at[pl.ds(s * RPT, RPT)], out.at[c, pl.ds(s * RPT, RPT)])


def _tc_mm1(x, W1):
    def body(x_ref, w1_ref, h1_ref):
        h1_ref[...] = jnp.dot(x_ref[...], w1_ref[...],
                              preferred_element_type=jnp.float32)

    return pl.pallas_call(
        body,
        out_shape=jax.ShapeDtypeStruct((N, H), jnp.float32),
    )(x, W1)


def _tc_scale(degp3, h1):
    def body(degp_ref, h1_ref, h1s_ref, dinv_ref):
        deg = degp_ref[0] + degp_ref[1] + 1.0
        dinv = 1.0 / jnp.sqrt(deg)
        dinv_ref[...] = dinv
        h1s_ref[...] = h1_ref[...] * dinv[:N]

    return pl.pallas_call(
        body,
        out_shape=(jax.ShapeDtypeStruct((N, H), jnp.float32),
                   jax.ShapeDtypeStruct((NPAD, 1), jnp.float32)),
    )(degp3, h1)


def _tc_mm2(p, h1s, dinv, b1, W2):
    def body(p_ref, h1s_ref, dinv_ref, b1_ref, w2_ref, g_ref):
        dinv = dinv_ref[...][:N]
        t = p_ref[0, :N, :] + p_ref[1, :N, :] + h1s_ref[...]
        h = jnp.maximum(t * dinv + b1_ref[...][None, :], 0.0)
        g_ref[...] = jnp.dot(h, w2_ref[...],
                             preferred_element_type=jnp.float32) * dinv

    return pl.pallas_call(
        body,
        out_shape=jax.ShapeDtypeStruct((N, CP), jnp.float32),
    )(p, h1s, dinv, b1, W2)


def _tc_finish(q, g, dinv, b2):
    def body(q_ref, g_ref, dinv_ref, b2_ref, out_ref):
        dinv = dinv_ref[...][:N]
        o = (q_ref[0, :N, :] + q_ref[1, :N, :] + g_ref[...]) * dinv
        o = o[:, :C] + b2_ref[...][None, :]
        m = jnp.max(o, axis=1, keepdims=True)
        lse = jnp.log(jnp.sum(jnp.exp(o - m), axis=1, keepdims=True)) + m
        out_ref[...] = o - lse

    return pl.pallas_call(
        body,
        out_shape=jax.ShapeDtypeStruct((N, C), jnp.float32),
    )(q, g, dinv, b2)


def kernel(x, edge_index, W1, b1, W2, b2):
    pad = EP - E
    src2d = jnp.concatenate(
        [edge_index[0], jnp.zeros((pad,), jnp.int32)]).reshape(EP // CH, CH)
    dst2d = jnp.concatenate(
        [edge_index[1], jnp.full((pad,), NPAD - 1, jnp.int32)]).reshape(EP // CH, CH)
    zeros_h = jnp.zeros((RPT, H), jnp.float32)
    zeros_c = jnp.zeros((RPT, CP), jnp.float32)
    W2p = jnp.pad(W2, ((0, 0), (0, CP - C)))
    zeros_1 = jnp.zeros((RPT,), jnp.float32)
    ones_ch = jnp.ones((CH,), jnp.float32)

    h1 = _tc_mm1(x, W1)
    degp = _sc_degree(dst2d, zeros_1, ones_ch)
    h1s, dinv = _tc_scale(degp.reshape(NC, NPAD, 1), h1)
    p = _propagate_h(h1s, src2d, dst2d, zeros_h)
    g = _tc_mm2(p, h1s, dinv, b1, W2p)
    q = _propagate_c(g, src2d, dst2d, zeros_c)
    return _tc_finish(q, g, dinv, b2)


# speedup vs baseline: 1.3656x; 1.1592x over previous
import functools

import jax
import jax.numpy as jnp
from jax import lax
from jax.experimental import pallas as pl
from jax.experimental.pallas import tpu as pltpu
from jax.experimental.pallas import tpu_sc as plsc

N = 10000
E = 320000
D = 128
H = 32
C = 2
CP = 8

NC = 2
NS = 16
NW = NC * NS
CH = 128
NCH = 80
KB = 16
F0 = 80
F1 = 2 * NCH - F0
SMAX = max(F0, F1)
EP = NW * NCH * CH
NPAD = 10240
RPT = NPAD // NS

_mesh = lambda: plsc.VectorSubcoreMesh(core_axis_name="c", subcore_axis_name="s")


def _make_propagate(width, dtype=jnp.float32):
    @functools.partial(
        pl.kernel,
        mesh=_mesh(),
        out_type=jax.ShapeDtypeStruct((NC, NPAD, width), dtype),
        compiler_params=pltpu.CompilerParams(use_tc_tiling_on_sc=False),
        scratch_types=[
            pltpu.VMEM((SMAX, CH), jnp.int32),
            pltpu.VMEM((SMAX, CH), jnp.int32),
        ]
        + [pltpu.VMEM((CH, width), dtype) for _ in range(KB)]
        + [pltpu.VMEM_SHARED((NPAD, width), dtype)]
        + [pltpu.SemaphoreType.DMA for _ in range(KB)],
    )
    def propagate(feat, src2d, dst2d, zeros, out, sidx, didx, *rest):
        rows = rest[:KB]
        acc = rest[KB]
        sems = rest[KB + 1:]
        c = lax.axis_index("c")
        s = lax.axis_index("s")

        @pl.when(c == 0)
        def _():
            pltpu.sync_copy(src2d.at[pl.ds(s * F0, F0)],
                            sidx.at[pl.ds(0, F0)])
            pltpu.sync_copy(dst2d.at[pl.ds(s * F0, F0)],
                            didx.at[pl.ds(0, F0)])

        @pl.when(c != 0)
        def _():
            pltpu.sync_copy(src2d.at[pl.ds(NS * F0 + s * F1, F1)],
                            sidx.at[pl.ds(0, F1)])
            pltpu.sync_copy(dst2d.at[pl.ds(NS * F0 + s * F1, F1)],
                            didx.at[pl.ds(0, F1)])

        pltpu.sync_copy(zeros, acc.at[pl.ds(s * RPT, RPT)])
        plsc.subcore_barrier()

        def block(j, carry):
            hs = [pltpu.async_copy(feat.at[sidx.at[j + b]], rows[b], sems[b])
                  for b in range(KB)]
            for b in range(KB):
                hs[b].wait()
                pltpu.sync_copy(rows[b], acc.at[didx.at[j + b]], add=True)
            return carry

        @pl.when(c == 0)
        def _():
            lax.fori_loop(0, F0 // KB, lambda k, cy: block(k * KB, cy), 0)

        @pl.when(c != 0)
        def _():
            lax.fori_loop(0, F1 // KB, lambda k, cy: block(k * KB, cy), 0)

        plsc.subcore_barrier()
        pltpu.sync_copy(acc.at[pl.ds(s * RPT, RPT)],
                        out.at[c, pl.ds(s * RPT, RPT)])

    return propagate


_propagate_h = _make_propagate(H, jnp.bfloat16)
_propagate_c = _make_propagate(CP)


@functools.partial(
    pl.kernel,
    mesh=_mesh(),
    out_type=jax.ShapeDtypeStruct((NC, NPAD), jnp.float32),
    compiler_params=pltpu.CompilerParams(use_tc_tiling_on_sc=False),
    scratch_types=[
        pltpu.VMEM((NCH, CH), jnp.int32),
        pltpu.VMEM((CH,), jnp.float32),
        pltpu.VMEM_SHARED((NPAD,), jnp.float32),
    ],
)
def _sc_degree(dst2d, zeros1, ones1, out, didx, ones_v, acc):
    c = lax.axis_index("c")
    s = lax.axis_index("s")
    wid = s * NC + c
    pltpu.sync_copy(zeros1, acc.at[pl.ds(s * RPT, RPT)])
    pltpu.sync_copy(dst2d.at[pl.ds(wid * NCH, NCH)], didx)
    pltpu.sync_copy(ones1, ones_v)
    plsc.subcore_barrier()

    def chunk(j, carry):
        pltpu.sync_copy(ones_v, acc.at[didx.at[j]], add=True)
        return carry

    lax.fori_loop(0, NCH, chunk, 0)
    plsc.subcore_barrier()
    pltpu.sync_copy(acc.at[pl.ds(s * RPT, RPT)], out.at[c, pl.ds(s * RPT, RPT)])


def _tc_mm1(x, W1):
    def body(x_ref, w1_ref, h1_ref):
        h1_ref[...] = jnp.dot(x_ref[...], w1_ref[...],
                              preferred_element_type=jnp.float32)

    return pl.pallas_call(
        body,
        out_shape=jax.ShapeDtypeStruct((N, H), jnp.float32),
    )(x, W1)


def _tc_scale(degp3, h1):
    def body(degp_ref, h1_ref, h1s_ref, hb_ref, dinv_ref):
        deg = degp_ref[0] + degp_ref[1] + 1.0
        dinv = 1.0 / jnp.sqrt(deg)
        dinv_ref[...] = dinv
        h1s = h1_ref[...] * dinv[:N]
        h1s_ref[...] = h1s
        hb_ref[...] = h1s.astype(jnp.bfloat16)

    return pl.pallas_call(
        body,
        out_shape=(jax.ShapeDtypeStruct((N, H), jnp.float32),
                   jax.ShapeDtypeStruct((N, H), jnp.bfloat16),
                   jax.ShapeDtypeStruct((NPAD, 1), jnp.float32)),
    )(degp3, h1)


def _tc_mm2(p, h1s, dinv, b1, W2):
    def body(p_ref, h1s_ref, dinv_ref, b1_ref, w2_ref, g_ref):
        dinv = dinv_ref[...][:N]
        acc = (p_ref[0, :N, :].astype(jnp.float32)
               + p_ref[1, :N, :].astype(jnp.float32))
        t = acc + h1s_ref[...]
        h = jnp.maximum(t * dinv + b1_ref[...][None, :], 0.0)
        g_ref[...] = jnp.dot(h, w2_ref[...],
                             preferred_element_type=jnp.float32) * dinv

    return pl.pallas_call(
        body,
        out_shape=jax.ShapeDtypeStruct((N, CP), jnp.float32),
    )(p, h1s, dinv, b1, W2)


def _tc_finish(q, g, dinv, b2):
    def body(q_ref, g_ref, dinv_ref, b2_ref, out_ref):
        dinv = dinv_ref[...][:N]
        o = (q_ref[0, :N, :] + q_ref[1, :N, :] + g_ref[...]) * dinv
        o = o[:, :C] + b2_ref[...][None, :]
        m = jnp.max(o, axis=1, keepdims=True)
        lse = jnp.log(jnp.sum(jnp.exp(o - m), axis=1, keepdims=True)) + m
        out_ref[...] = o - lse

    return pl.pallas_call(
        body,
        out_shape=jax.ShapeDtypeStruct((N, C), jnp.float32),
    )(q, g, dinv, b2)


def kernel(x, edge_index, W1, b1, W2, b2):
    pad = EP - E
    src2d = jnp.concatenate(
        [edge_index[0], jnp.zeros((pad,), jnp.int32)]).reshape(EP // CH, CH)
    dst2d = jnp.concatenate(
        [edge_index[1], jnp.full((pad,), NPAD - 1, jnp.int32)]).reshape(EP // CH, CH)
    zeros_h = jnp.zeros((RPT, H), jnp.bfloat16)
    zeros_c = jnp.zeros((RPT, CP), jnp.float32)
    W2p = jnp.pad(W2, ((0, 0), (0, CP - C)))
    zeros_1 = jnp.zeros((RPT,), jnp.float32)
    ones_ch = jnp.ones((CH,), jnp.float32)

    h1 = _tc_mm1(x, W1)
    degp = _sc_degree(dst2d, zeros_1, ones_ch)
    h1s, h1b, dinv = _tc_scale(degp.reshape(NC, NPAD, 1), h1)
    p = _propagate_h(h1b, src2d, dst2d, zeros_h)
    g = _tc_mm2(p, h1s, dinv, b1, W2p)
    q = _propagate_c(g, src2d, dst2d, zeros_c)
    return _tc_finish(q, g, dinv, b2)
